# Initial kernel scaffold; baseline (speedup 1.0000x reference)
import jax
import jax.numpy as jnp
from jax.experimental import pallas as pl


def kernel(p1, x1, o1, p2, x2, o2, W1_feat, W1_dir, bn1_gamma, bn1_beta, W2_feat, W2_dir, bn2_gamma, bn2_beta):
    raise NotImplementedError("write your pallas kernel here")


# speedup vs baseline: 10.8407x; 10.8407x over previous
import functools

import jax
import jax.numpy as jnp
from jax import lax
from jax.experimental import pallas as pl
from jax.experimental.pallas import tpu as pltpu
from jax.experimental.pallas import tpu_sc as plsc

EPS = 1e-6

N1, N2 = 8192, 2048
CO = 64
FD = 3 * CO

_NC, _NS, _L = 2, 16, 16
_NW = _NC * _NS
_QPW = N1 // _NW
_CHUNK = 64
_NCHUNK = _QPW // _CHUNK
_FDP = 256


def _vn_body(n, x_ref, wf_ref, wd_ref, g_ref, b_ref, y_ref):
    x = x_ref[...]
    p = jnp.dot(wf_ref[...], x, preferred_element_type=jnp.float32)
    d = jnp.dot(wd_ref[...], x, preferred_element_type=jnp.float32)
    pv = [p[:, v * n:(v + 1) * n] for v in range(3)]
    dv = [d[:, v * n:(v + 1) * n] for v in range(3)]
    nrm = jnp.sqrt(pv[0] * pv[0] + pv[1] * pv[1] + pv[2] * pv[2]) + EPS
    mean = jnp.mean(nrm, axis=1, keepdims=True)
    cen = nrm - mean
    var = jnp.mean(cen * cen, axis=1, keepdims=True)
    nbn = cen / jnp.sqrt(var + 1e-5) * g_ref[...] + b_ref[...]
    scale = nbn / nrm
    dotp = (pv[0] * dv[0] + pv[1] * dv[1] + pv[2] * dv[2]) * scale
    dnsq = dv[0] * dv[0] + dv[1] * dv[1] + dv[2] * dv[2]
    coef = jnp.where(dotp < 0, 0.8 * dotp / (dnsq + EPS), 0.0)
    for v in range(3):
        y_ref[:, v * n:(v + 1) * n] = pv[v] * scale - coef * dv[v]


def _vn(x, wf, wd, g, b, n):
    co, cin = wf.shape
    return pl.pallas_call(
        functools.partial(_vn_body, n),
        out_shape=jax.ShapeDtypeStruct((co, 3 * n), jnp.float32),
    )(x, wf, wd, g.reshape(co, 1), b.reshape(co, 1))


_BQ = 256
_IDXM = 0x7FF


def _knn_body(p2_ref, p1t_ref, idx_ref, w_ref):
    p2 = p2_ref[...]
    p1 = p1t_ref[...]
    dvs = []
    for v in range(3):
        dv = p2[:, v:v + 1] - p1[v:v + 1, :]
        dvs.append(dv * dv)
    d2 = (dvs[0] + dvs[1]) + dvs[2]
    iota = lax.broadcasted_iota(jnp.int32, (N2, _BQ), 0)
    inf = jnp.float32(jnp.inf)
    recips = []
    for k in range(3):
        mk = jnp.min(d2, axis=0, keepdims=True)
        idxk = jnp.min(jnp.where(d2 == mk, iota, N2), axis=0, keepdims=True)
        idx_ref[k:k + 1, :] = idxk
        if k < 2:
            d2 = jnp.where(iota == idxk, inf, d2)
        recips.append(1.0 / (jnp.sqrt(jnp.maximum(mk, 0.0)) + 1e-8))
    rs = recips[0] + recips[1] + recips[2]
    for k in range(3):
        w_ref[k:k + 1, :] = recips[k] / rs


def _knn(p2, p1t):
    return pl.pallas_call(
        _knn_body,
        grid=(N1 // _BQ,),
        in_specs=[
            pl.BlockSpec((N2, 3), lambda i: (0, 0)),
            pl.BlockSpec((3, _BQ), lambda i: (0, i)),
        ],
        out_specs=[
            pl.BlockSpec((3, _BQ), lambda i: (0, i)),
            pl.BlockSpec((3, _BQ), lambda i: (0, i)),
        ],
        out_shape=[
            jax.ShapeDtypeStruct((3, N1), jnp.int32),
            jax.ShapeDtypeStruct((3, N1), jnp.float32),
        ],
    )(p2, p1t)


def _interp_body(feat_hbm, idx_hbm, w_hbm, out_hbm,
                 i0_v, i1_v, i2_v, w_v, r0_v, r1_v, r2_v, out_v, sem):
    wid = lax.axis_index("s") * _NC + lax.axis_index("c")
    for c in range(_NCHUNK):
        qbase = wid * _QPW + c * _CHUNK
        pltpu.sync_copy(idx_hbm.at[pl.ds(qbase, _CHUNK)], i0_v)
        pltpu.sync_copy(idx_hbm.at[pl.ds(N1 + qbase, _CHUNK)], i1_v)
        pltpu.sync_copy(idx_hbm.at[pl.ds(2 * N1 + qbase, _CHUNK)], i2_v)
        pltpu.sync_copy(w_hbm.at[pl.ds(qbase, _CHUNK)],
                        w_v.at[0, pl.ds(0, _CHUNK)])
        pltpu.sync_copy(w_hbm.at[pl.ds(N1 + qbase, _CHUNK)],
                        w_v.at[1, pl.ds(0, _CHUNK)])
        pltpu.sync_copy(w_hbm.at[pl.ds(2 * N1 + qbase, _CHUNK)],
                        w_v.at[2, pl.ds(0, _CHUNK)])
        cp0 = pltpu.async_copy(feat_hbm.at[i0_v], r0_v, sem)
        cp1 = pltpu.async_copy(feat_hbm.at[i1_v], r1_v, sem)
        cp2 = pltpu.async_copy(feat_hbm.at[i2_v], r2_v, sem)
        cp0.wait()
        cp1.wait()
        cp2.wait()

        def body(i, carry):
            w0 = w_v[0, pl.ds(i, _L)][0]
            w1 = w_v[1, pl.ds(i, _L)][0]
            w2 = w_v[2, pl.ds(i, _L)][0]
            for f in range(FD // _L):
                sl = pl.ds(f * _L, _L)
                out_v[i, sl] = (w0 * r0_v[i, sl] + w1 * r1_v[i, sl]
                                + w2 * r2_v[i, sl])
            return carry

        lax.fori_loop(0, _CHUNK, body, 0)
        pltpu.sync_copy(out_v, out_hbm.at[pl.ds(qbase, _CHUNK)])


def _interp(feat, idx_flat, w_flat):
    mesh = plsc.VectorSubcoreMesh(core_axis_name="c", subcore_axis_name="s")
    return pl.kernel(
        _interp_body,
        out_type=jax.ShapeDtypeStruct((N1, FD), jnp.float32),
        mesh=mesh,
        scratch_types=[
            pltpu.VMEM((_CHUNK,), jnp.int32),
            pltpu.VMEM((_CHUNK,), jnp.int32),
            pltpu.VMEM((_CHUNK,), jnp.int32),
            pltpu.VMEM((3, _CHUNK + _L), jnp.float32),
            pltpu.VMEM((_CHUNK, _FDP), jnp.float32),
            pltpu.VMEM((_CHUNK, _FDP), jnp.float32),
            pltpu.VMEM((_CHUNK, _FDP), jnp.float32),
            pltpu.VMEM((_CHUNK, FD), jnp.float32),
            pltpu.SemaphoreType.DMA,
        ],
    )(feat, idx_flat, w_flat)


def _add_body(ir_ref, y1_ref, o_ref):
    o_ref[...] = y1_ref[...] + jnp.transpose(ir_ref[...], (1, 0))


def _add(interp_rows, y1_192):
    return pl.pallas_call(
        _add_body,
        out_shape=jax.ShapeDtypeStruct((FD, N1), jnp.float32),
    )(interp_rows, y1_192)


def kernel(p1, x1, o1, p2, x2, o2, W1_feat, W1_dir, bn1_gamma, bn1_beta,
           W2_feat, W2_dir, bn2_gamma, bn2_beta):
    y1 = _vn(x1.reshape(CO, 3 * N1), W1_feat, W1_dir,
             bn1_gamma, bn1_beta, N1)
    y2 = _vn(x2.reshape(128, 3 * N2), W2_feat, W2_dir,
             bn2_gamma, bn2_beta, N2)
    feat = jnp.transpose(y2.reshape(CO, 3, N2), (2, 0, 1)).reshape(N2, FD)
    feat = jnp.pad(feat, ((0, 0), (0, _FDP - FD)))
    idx, w = _knn(p2, jnp.transpose(p1, (1, 0)))
    interp_rows = _interp(feat, idx.reshape(3 * N1), w.reshape(3 * N1))
    out = _add(interp_rows, y1.reshape(FD, N1))
    return out.reshape(1, CO, 3, N1)


# speedup vs baseline: 12.3884x; 1.1428x over previous
import functools

import jax
import jax.numpy as jnp
from jax import lax
from jax.experimental import pallas as pl
from jax.experimental.pallas import tpu as pltpu
from jax.experimental.pallas import tpu_sc as plsc

EPS = 1e-6

N1, N2 = 8192, 2048
CO = 64
FD = 3 * CO

_NC, _NS, _L = 2, 16, 16
_NW = _NC * _NS
_QPW = N1 // _NW
_CHUNK = 64
_NCHUNK = _QPW // _CHUNK
_FDP = 256


def _vn_math(xv, wf_ref, wd_ref, g_ref, b_ref):
    pv = [jnp.dot(wf_ref[...], x, preferred_element_type=jnp.float32)
          for x in xv]
    dv = [jnp.dot(wd_ref[...], x, preferred_element_type=jnp.float32)
          for x in xv]
    nrm = jnp.sqrt(pv[0] * pv[0] + pv[1] * pv[1] + pv[2] * pv[2]) + EPS
    mean = jnp.mean(nrm, axis=1, keepdims=True)
    cen = nrm - mean
    var = jnp.mean(cen * cen, axis=1, keepdims=True)
    nbn = cen / jnp.sqrt(var + 1e-5) * g_ref[...] + b_ref[...]
    scale = nbn / nrm
    dotp = (pv[0] * dv[0] + pv[1] * dv[1] + pv[2] * dv[2]) * scale
    dnsq = dv[0] * dv[0] + dv[1] * dv[1] + dv[2] * dv[2]
    coef = jnp.where(dotp < 0, 0.8 * dotp / (dnsq + EPS), 0.0)
    return [pv[v] * scale - coef * dv[v] for v in range(3)]


def _vn1_body(x_ref, wf_ref, wd_ref, g_ref, b_ref, y_ref):
    xv = [x_ref[:, v, :] for v in range(3)]
    yv = _vn_math(xv, wf_ref, wd_ref, g_ref, b_ref)
    for v in range(3):
        y_ref[:, v * N1:(v + 1) * N1] = yv[v]


def _vn1(x, wf, wd, g, b):
    return pl.pallas_call(
        _vn1_body,
        out_shape=jax.ShapeDtypeStruct((CO, 3 * N1), jnp.float32),
    )(x, wf, wd, g.reshape(CO, 1), b.reshape(CO, 1))


def _vn2_body(x_ref, wf_ref, wd_ref, g_ref, b_ref, f_ref):
    xv = [x_ref[:, v, :] for v in range(3)]
    yv = _vn_math(xv, wf_ref, wd_ref, g_ref, b_ref)
    for v in range(3):
        f_ref[:, v * CO:(v + 1) * CO] = jnp.transpose(yv[v], (1, 0))
    f_ref[:, FD:_FDP] = jnp.zeros((N2, _FDP - FD), jnp.float32)


def _vn2(x, wf, wd, g, b):
    return pl.pallas_call(
        _vn2_body,
        out_shape=jax.ShapeDtypeStruct((N2, _FDP), jnp.float32),
    )(x, wf, wd, g.reshape(CO, 1), b.reshape(CO, 1))


_BQ = 256


def _knn_body(p2_ref, p1t_ref, idx_ref, w_ref):
    p2 = p2_ref[...]
    p1 = p1t_ref[...]
    dvs = []
    for v in range(3):
        dv = p2[:, v:v + 1] - p1[v:v + 1, :]
        dvs.append(dv * dv)
    d2 = (dvs[0] + dvs[1]) + dvs[2]
    iota = lax.broadcasted_iota(jnp.int32, (N2, _BQ), 0)
    inf = jnp.float32(jnp.inf)
    recips = []
    for k in range(3):
        mk = jnp.min(d2, axis=0, keepdims=True)
        idxk = jnp.min(jnp.where(d2 == mk, iota, N2), axis=0, keepdims=True)
        idx_ref[k:k + 1, :] = idxk
        if k < 2:
            d2 = jnp.where(iota == idxk, inf, d2)
        recips.append(1.0 / (jnp.sqrt(jnp.maximum(mk, 0.0)) + 1e-8))
    rs = recips[0] + recips[1] + recips[2]
    for k in range(3):
        w_ref[k:k + 1, :] = recips[k] / rs


def _knn(p2, p1t):
    return pl.pallas_call(
        _knn_body,
        grid=(N1 // _BQ,),
        in_specs=[
            pl.BlockSpec((N2, 3), lambda i: (0, 0)),
            pl.BlockSpec((3, _BQ), lambda i: (0, i)),
        ],
        out_specs=[
            pl.BlockSpec((3, _BQ), lambda i: (0, i)),
            pl.BlockSpec((3, _BQ), lambda i: (0, i)),
        ],
        out_shape=[
            jax.ShapeDtypeStruct((3, N1), jnp.int32),
            jax.ShapeDtypeStruct((3, N1), jnp.float32),
        ],
    )(p2, p1t)


def _interp_body(feat_hbm, idx_hbm, w_hbm, out_hbm,
                 i0_v, i1_v, i2_v, w_v, r0_v, r1_v, r2_v, out_v, sem):
    wid = lax.axis_index("s") * _NC + lax.axis_index("c")
    for c in range(_NCHUNK):
        qbase = wid * _QPW + c * _CHUNK
        pltpu.sync_copy(idx_hbm.at[0, pl.ds(qbase, _CHUNK)], i0_v)
        pltpu.sync_copy(idx_hbm.at[1, pl.ds(qbase, _CHUNK)], i1_v)
        pltpu.sync_copy(idx_hbm.at[2, pl.ds(qbase, _CHUNK)], i2_v)
        pltpu.sync_copy(w_hbm.at[0, pl.ds(qbase, _CHUNK)],
                        w_v.at[0, pl.ds(0, _CHUNK)])
        pltpu.sync_copy(w_hbm.at[1, pl.ds(qbase, _CHUNK)],
                        w_v.at[1, pl.ds(0, _CHUNK)])
        pltpu.sync_copy(w_hbm.at[2, pl.ds(qbase, _CHUNK)],
                        w_v.at[2, pl.ds(0, _CHUNK)])
        cp0 = pltpu.async_copy(feat_hbm.at[i0_v], r0_v, sem)
        cp1 = pltpu.async_copy(feat_hbm.at[i1_v], r1_v, sem)
        cp2 = pltpu.async_copy(feat_hbm.at[i2_v], r2_v, sem)
        cp0.wait()
        cp1.wait()
        cp2.wait()

        def body(i, carry):
            w0 = w_v[0, pl.ds(i, _L)][0]
            w1 = w_v[1, pl.ds(i, _L)][0]
            w2 = w_v[2, pl.ds(i, _L)][0]
            for f in range(FD // _L):
                sl = pl.ds(f * _L, _L)
                out_v[i, sl] = (w0 * r0_v[i, sl] + w1 * r1_v[i, sl]
                                + w2 * r2_v[i, sl])
            return carry

        lax.fori_loop(0, _CHUNK, body, 0)
        pltpu.sync_copy(out_v, out_hbm.at[pl.ds(qbase, _CHUNK)])


def _interp(feat, idx, w):
    mesh = plsc.VectorSubcoreMesh(core_axis_name="c", subcore_axis_name="s")
    return pl.kernel(
        _interp_body,
        out_type=jax.ShapeDtypeStruct((N1, FD), jnp.float32),
        mesh=mesh,
        scratch_types=[
            pltpu.VMEM((_CHUNK,), jnp.int32),
            pltpu.VMEM((_CHUNK,), jnp.int32),
            pltpu.VMEM((_CHUNK,), jnp.int32),
            pltpu.VMEM((3, _CHUNK + _L), jnp.float32),
            pltpu.VMEM((_CHUNK, _FDP), jnp.float32),
            pltpu.VMEM((_CHUNK, _FDP), jnp.float32),
            pltpu.VMEM((_CHUNK, _FDP), jnp.float32),
            pltpu.VMEM((_CHUNK, FD), jnp.float32),
            pltpu.SemaphoreType.DMA,
        ],
    )(feat, idx, w)


def _add_body(ir_ref, y1_ref, o_ref):
    for v in range(3):
        o_ref[:, v, :] = (
            y1_ref[:, v * N1:(v + 1) * N1]
            + jnp.transpose(ir_ref[:, v * CO:(v + 1) * CO], (1, 0)))


def _add(interp_rows, y1):
    return pl.pallas_call(
        _add_body,
        out_shape=jax.ShapeDtypeStruct((CO, 3, N1), jnp.float32),
    )(interp_rows, y1)


def kernel(p1, x1, o1, p2, x2, o2, W1_feat, W1_dir, bn1_gamma, bn1_beta,
           W2_feat, W2_dir, bn2_gamma, bn2_beta):
    y1 = _vn1(x1.reshape(CO, 3, N1), W1_feat, W1_dir,
              bn1_gamma, bn1_beta)
    feat = _vn2(x2.reshape(128, 3, N2), W2_feat, W2_dir,
                bn2_gamma, bn2_beta)
    idx, w = _knn(p2, jnp.transpose(p1, (1, 0)))
    interp_rows = _interp(feat, idx, w)
    return _add(interp_rows, y1).reshape(1, CO, 3, N1)


# speedup vs baseline: 14.2769x; 1.1524x over previous
import jax
import jax.numpy as jnp
from jax import lax
from jax.experimental import pallas as pl
from jax.experimental.pallas import tpu as pltpu
from jax.experimental.pallas import tpu_sc as plsc

EPS = 1e-6

N1, N2 = 8192, 2048
CO = 64
FD = 3 * CO

_NC, _NS, _L = 2, 16, 16
_NW = _NC * _NS
_QPW = N1 // _NW
_CHUNK = 64
_NCHUNK = _QPW // _CHUNK
_FDP = 256


def _vn_math(x_ref, wf_ref, wd_ref, g_ref, b_ref):
    xv = [x_ref[v] for v in range(3)]
    pv = [jnp.dot(wf_ref[...], x, preferred_element_type=jnp.float32)
          for x in xv]
    dv = [jnp.dot(wd_ref[...], x, preferred_element_type=jnp.float32)
          for x in xv]
    nrm = jnp.sqrt(pv[0] * pv[0] + pv[1] * pv[1] + pv[2] * pv[2]) + EPS
    mean = jnp.mean(nrm, axis=1, keepdims=True)
    cen = nrm - mean
    var = jnp.mean(cen * cen, axis=1, keepdims=True)
    nbn = cen / jnp.sqrt(var + 1e-5) * g_ref[...] + b_ref[...]
    scale = nbn / nrm
    dotp = (pv[0] * dv[0] + pv[1] * dv[1] + pv[2] * dv[2]) * scale
    dnsq = dv[0] * dv[0] + dv[1] * dv[1] + dv[2] * dv[2]
    coef = jnp.where(dotp < 0, 0.8 * dotp / (dnsq + EPS), 0.0)
    return [pv[v] * scale - coef * dv[v] for v in range(3)]


def _vn1_body(x_ref, wf_ref, wd_ref, g_ref, b_ref, y_ref):
    yv = _vn_math(x_ref, wf_ref, wd_ref, g_ref, b_ref)
    for v in range(3):
        y_ref[v] = yv[v]


def _vn1(xt, wf, wd, g, b):
    return pl.pallas_call(
        _vn1_body,
        out_shape=jax.ShapeDtypeStruct((3, CO, N1), jnp.float32),
    )(xt, wf, wd, g.reshape(CO, 1), b.reshape(CO, 1))


def _vn2_body(x_ref, wf_ref, wd_ref, g_ref, b_ref, f_ref):
    yv = _vn_math(x_ref, wf_ref, wd_ref, g_ref, b_ref)
    for v in range(3):
        f_ref[:, v * CO:(v + 1) * CO] = jnp.transpose(yv[v], (1, 0))
    f_ref[:, FD:_FDP] = jnp.zeros((N2, _FDP - FD), jnp.float32)


def _vn2(xt, wf, wd, g, b):
    return pl.pallas_call(
        _vn2_body,
        out_shape=jax.ShapeDtypeStruct((N2, _FDP), jnp.float32),
    )(xt, wf, wd, g.reshape(CO, 1), b.reshape(CO, 1))


_BQ = 256


def _knn_body(p2_ref, p1t_ref, idx_ref, w_ref):
    p2 = p2_ref[...]
    p1 = p1t_ref[...]
    dvs = []
    for v in range(3):
        dv = p2[:, v:v + 1] - p1[v:v + 1, :]
        dvs.append(dv * dv)
    d2 = (dvs[0] + dvs[1]) + dvs[2]
    iota = lax.broadcasted_iota(jnp.int32, (N2, _BQ), 0)
    inf = jnp.float32(jnp.inf)
    recips = []
    for k in range(3):
        mk = jnp.min(d2, axis=0, keepdims=True)
        idxk = jnp.min(jnp.where(d2 == mk, iota, N2), axis=0, keepdims=True)
        idx_ref[k:k + 1, :] = idxk
        if k < 2:
            d2 = jnp.where(iota == idxk, inf, d2)
        recips.append(1.0 / (jnp.sqrt(jnp.maximum(mk, 0.0)) + 1e-8))
    rs = recips[0] + recips[1] + recips[2]
    for k in range(3):
        w_ref[k:k + 1, :] = recips[k] / rs


def _knn(p2, p1t):
    return pl.pallas_call(
        _knn_body,
        grid=(N1 // _BQ,),
        in_specs=[
            pl.BlockSpec((N2, 3), lambda i: (0, 0)),
            pl.BlockSpec((3, _BQ), lambda i: (0, i)),
        ],
        out_specs=[
            pl.BlockSpec((3, _BQ), lambda i: (0, i)),
            pl.BlockSpec((3, _BQ), lambda i: (0, i)),
        ],
        out_shape=[
            jax.ShapeDtypeStruct((3, N1), jnp.int32),
            jax.ShapeDtypeStruct((3, N1), jnp.float32),
        ],
    )(p2, p1t)


def _interp_body(feat_hbm, idx_hbm, w_hbm, out_hbm,
                 i0a, i1a, i2a, i0b, i1b, i2b,
                 r0a, r1a, r2a, r0b, r1b, r2b,
                 w_v, out_v, sema, semb):
    wid = lax.axis_index("s") * _NC + lax.axis_index("c")
    ibufs = [(i0a, i1a, i2a), (i0b, i1b, i2b)]
    rbufs = [(r0a, r1a, r2a), (r0b, r1b, r2b)]
    sems = [sema, semb]

    def stage(c):
        b = c % 2
        qbase = wid * _QPW + c * _CHUNK
        iv, rv = ibufs[b], rbufs[b]
        for k in range(3):
            pltpu.sync_copy(idx_hbm.at[k, pl.ds(qbase, _CHUNK)], iv[k])
            pltpu.sync_copy(w_hbm.at[k, pl.ds(qbase, _CHUNK)],
                            w_v.at[b, k, pl.ds(0, _CHUNK)])
        return [pltpu.async_copy(feat_hbm.at[iv[k]], rv[k], sems[b])
                for k in range(3)]

    cps = stage(0)
    for c in range(_NCHUNK):
        b = c % 2
        nxt = stage(c + 1) if c + 1 < _NCHUNK else None
        for cp in cps:
            cp.wait()
        cps = nxt
        rv = rbufs[b]
        qbase = wid * _QPW + c * _CHUNK

        def body(i, carry):
            w0 = w_v[b, 0, pl.ds(i, _L)][0]
            w1 = w_v[b, 1, pl.ds(i, _L)][0]
            w2 = w_v[b, 2, pl.ds(i, _L)][0]
            for f in range(FD // _L):
                sl = pl.ds(f * _L, _L)
                out_v[i, sl] = (w0 * rv[0][i, sl] + w1 * rv[1][i, sl]
                                + w2 * rv[2][i, sl])
            return carry

        lax.fori_loop(0, _CHUNK, body, 0)
        pltpu.sync_copy(out_v, out_hbm.at[pl.ds(qbase, _CHUNK)])


def _interp(feat, idx, w):
    mesh = plsc.VectorSubcoreMesh(core_axis_name="c", subcore_axis_name="s")
    ivmem = [pltpu.VMEM((_CHUNK,), jnp.int32) for _ in range(6)]
    rvmem = [pltpu.VMEM((_CHUNK, _FDP), jnp.float32) for _ in range(6)]
    return pl.kernel(
        _interp_body,
        out_type=jax.ShapeDtypeStruct((N1, FD), jnp.float32),
        mesh=mesh,
        scratch_types=ivmem + rvmem + [
            pltpu.VMEM((2, 3, _CHUNK + _L), jnp.float32),
            pltpu.VMEM((_CHUNK, FD), jnp.float32),
            pltpu.SemaphoreType.DMA,
            pltpu.SemaphoreType.DMA,
        ],
    )(feat, idx, w)


def _add_body(ir_ref, y1_ref, o_ref):
    for v in range(3):
        o_ref[v] = y1_ref[v] + jnp.transpose(
            ir_ref[:, v * CO:(v + 1) * CO], (1, 0))


def _add(interp_rows, y1):
    return pl.pallas_call(
        _add_body,
        out_shape=jax.ShapeDtypeStruct((3, CO, N1), jnp.float32),
    )(interp_rows, y1)


def kernel(p1, x1, o1, p2, x2, o2, W1_feat, W1_dir, bn1_gamma, bn1_beta,
           W2_feat, W2_dir, bn2_gamma, bn2_beta):
    x1t = jnp.transpose(x1.reshape(CO, 3, N1), (1, 0, 2))
    x2t = jnp.transpose(x2.reshape(128, 3, N2), (1, 0, 2))
    feat = _vn2(x2t, W2_feat, W2_dir, bn2_gamma, bn2_beta)
    y1 = _vn1(x1t, W1_feat, W1_dir, bn1_gamma, bn1_beta)
    p1t = jnp.transpose(p1, (1, 0))
    p1t = lax.optimization_barrier((p1t, feat))[0]
    idx, w = _knn(p2, p1t)
    interp_rows = _interp(feat, idx, w)
    out = _add(interp_rows, y1)
    return jnp.transpose(out, (1, 0, 2)).reshape(1, CO, 3, N1)
